# baseline (speedup 1.0000x reference)
import jax
import jax.numpy as jnp
from jax.experimental import pallas as pl
from jax.experimental.pallas import tpu as pltpu

X_MIN, X_MAX = 0.0, 200.0
Y_MIN, Y_MAX = -50.0, 50.0
S = 0.4
INV_S = 2.5
ALPHA = 0.25
W_CLS, W_REG, W_DIR = 1.0, 2.0, 0.2
CW = (1.0, 5.0, 5.0)
HT = 25
H_FULL = 250


def _ls_pair(x):
    sp = jnp.log1p(jnp.exp(-jnp.abs(x)))
    ls_pos = jnp.where(x >= 0, -sp, x - sp)
    return ls_pos, ls_pos - x


def _loss_kernel(cls_ref, reg_ref, dir_ref, gt_ref, out_ref, cmask_ref, acc_ref):
    f32 = jnp.float32
    B = gt_ref.shape[0]
    N = gt_ref.shape[2]
    W = cls_ref.shape[3]
    t = pl.program_id(0)
    nt = pl.num_programs(0)

    dn_rows = (((1,), (1,)), ((), ()))
    dn_gram = (((0,), (0,)), ((), ()))

    def dot_nt(a, b):
        return jax.lax.dot_general(a, b, dn_rows, preferred_element_type=f32)

    def gram(a, b):
        return jax.lax.dot_general(a, b, dn_gram, preferred_element_type=f32)

    gt = gt_ref[...]
    x, y = gt[:, 0, :], gt[:, 1, :]
    rot = gt[:, 6, :]
    cid = gt[:, 7, :].astype(jnp.int32)
    gxf = jnp.floor((x - X_MIN) * INV_S)
    gyf = jnp.floor((y - Y_MIN) * INV_S)
    gx = gxf.astype(jnp.int32)
    gy = gyf.astype(jnp.int32)
    valid = ((x >= X_MIN) & (x < X_MAX) & (y >= Y_MIN) & (y < Y_MAX)
             & (gx >= 0) & (gx < W) & (gy >= 0) & (gy < H_FULL))
    dbin0 = jnp.cos(rot) >= 0.0

    @pl.when(t == 0)
    def _():
        z3, l3, w3, h3 = gt[:, 2, :], gt[:, 3, :], gt[:, 4, :], gt[:, 5, :]
        rv_all = ((x - (X_MIN + (gxf + 0.5) * S)) * INV_S,
                  (y - (Y_MIN + (gyf + 0.5) * S)) * INV_S, z3,
                  jnp.log(jnp.maximum(l3, 1e-3)),
                  jnp.log(jnp.maximum(w3, 1e-3)),
                  jnp.log(jnp.maximum(h3, 1e-3)), jnp.sin(rot))
        col_io = jax.lax.broadcasted_iota(jnp.int32, (W, N), 0)
        row_io = jax.lax.broadcasted_iota(jnp.int32, (H_FULL, N), 0)
        later = jax.lax.broadcasted_iota(jnp.int32, (N, N), 0) > \
            jax.lax.broadcasted_iota(jnp.int32, (N, N), 1)
        for b in range(B):
            gxb, gyb, vb = gx[b:b + 1, :], gy[b:b + 1, :], valid[b:b + 1, :]
            cols_eq = col_io == gxb
            cols_hit = jnp.where(cols_eq & vb, 1.0, 0.0).astype(f32)
            cmask_ref[b, 0] = cols_hit
            cmask_ref[b, 1] = jnp.where((jnp.abs(col_io - gxb) <= 1) & vb,
                                        1.0, 0.0).astype(f32)
            rows_oh = jnp.where((row_io == gyb) & vb, 1.0, 0.0).astype(f32)
            same = (gram(rows_oh, rows_oh) > 0.0) & \
                (gram(cols_hit, cols_hit) > 0.0)
            killed = jnp.any(same & later, axis=0, keepdims=True)
            islast = vb & jnp.logical_not(killed)
            keep = cols_eq & islast
            for i in range(7):
                cmask_ref[b, 2 + i] = jnp.where(keep, rv_all[i][b:b + 1, :],
                                                0.0).astype(f32)
        acc_ref[...] = jnp.zeros_like(acc_ref)

    row_io_t = jax.lax.broadcasted_iota(jnp.int32, (HT, N), 0)
    rbase = t * HT

    cls_acc = jnp.zeros((), f32)
    vm_cnt = jnp.zeros((), f32)
    reg_num = jnp.zeros((), f32)
    pos_cnt = jnp.zeros((), f32)
    dir_acc = jnp.zeros((), f32)

    for b in range(B):
        gy_rel = gy[b:b + 1, :] - rbase
        rows_eq = row_io_t == gy_rel
        rows_near = jnp.abs(row_io_t - gy_rel) <= 1
        cid_b = cid[b:b + 1, :]
        cols_hit = cmask_ref[b, 0]

        poscnt = None
        for c in range(3):
            mc = cid_b == c
            hitcnt = dot_nt(jnp.where(rows_eq & mc, 1.0, 0.0).astype(f32),
                            cols_hit)
            nearcnt = dot_nt(jnp.where(rows_near & mc, 1.0, 0.0).astype(f32),
                             cmask_ref[b, 1])
            hit = hitcnt > 0.0
            vm = hit | (nearcnt == 0.0)
            xl = cls_ref[c, :, b, :]
            ls_pos, ls_neg = _ls_pair(xl)
            ls_a = jnp.where(hit, ls_pos, ls_neg)
            ls_b = jnp.where(hit, ls_neg, ls_pos)
            fac = jnp.where(hit, ALPHA * CW[c], 1.0 - ALPHA)
            term = fac * jnp.exp(3.0 * ls_b) * ls_a
            cls_acc = cls_acc + jnp.sum(jnp.where(vm, term, 0.0))
            vm_cnt = vm_cnt + jnp.sum(jnp.where(vm, 1.0, 0.0))
            poscnt = hitcnt if poscnt is None else poscnt + hitcnt

        pos = poscnt > 0.0
        pos_cnt = pos_cnt + jnp.sum(jnp.where(pos, 1.0, 0.0))

        rows_oh = jnp.where(rows_eq, 1.0, 0.0).astype(f32)
        for i in range(7):
            reg_t = dot_nt(rows_oh, cmask_ref[b, 2 + i])
            d = reg_ref[i, :, b, :] - reg_t
            ad = jnp.abs(d)
            sl1 = jnp.where(ad < 1.0, 0.5 * d * d, ad - 0.5)
            reg_num = reg_num + jnp.sum(jnp.where(pos, sl1, 0.0))

        for k in range(2):
            mk = dbin0[b:b + 1, :] if k == 0 else \
                jnp.logical_not(dbin0[b:b + 1, :])
            tk = dot_nt(jnp.where(rows_eq & mk, 1.0, 0.0).astype(f32),
                        cols_hit) > 0.0
            ls_pos, ls_neg = _ls_pair(dir_ref[b, :, k, :])
            dsel = jnp.where(tk, ls_pos, ls_neg)
            dir_acc = dir_acc + jnp.sum(jnp.where(pos, dsel, 0.0))

    lane = jax.lax.broadcasted_iota(jnp.int32, (1, 128), 1)
    vec = (jnp.where(lane == 0, cls_acc, 0.0)
           + jnp.where(lane == 1, vm_cnt, 0.0)
           + jnp.where(lane == 2, reg_num, 0.0)
           + jnp.where(lane == 3, pos_cnt, 0.0)
           + jnp.where(lane == 4, dir_acc, 0.0)).astype(f32)
    acc_ref[...] = acc_ref[...] + vec

    @pl.when(t == nt - 1)
    def _():
        s = acc_ref[...]
        cls_loss = -s[0, 0] / (s[0, 1] + 1e-6)
        rcnt = s[0, 3] * 7.0
        reg_loss = jnp.where(rcnt > 0, s[0, 2] / jnp.maximum(rcnt, 1.0), 0.0)
        dcnt = s[0, 3] * 2.0
        dir_loss = jnp.where(dcnt > 0, -s[0, 4] / jnp.maximum(dcnt, 1.0), 0.0)
        total = W_CLS * cls_loss + W_REG * reg_loss + W_DIR * dir_loss
        l4 = jax.lax.broadcasted_iota(jnp.int32, (1, 1, 4), 2)
        out_ref[...] = jnp.where(
            l4 == 0, total,
            jnp.where(l4 == 1, cls_loss,
                      jnp.where(l4 == 2, reg_loss, dir_loss))).astype(f32)


def kernel(cls_pred, reg_pred, dir_pred, gt_boxes, batch_size):
    B, C, H, W = cls_pred.shape
    N = gt_boxes.shape[1]
    cls_t = jnp.transpose(cls_pred, (1, 2, 0, 3))
    reg_t = jnp.transpose(reg_pred, (1, 2, 0, 3))
    dir_t = jnp.transpose(dir_pred, (0, 2, 1, 3))
    gt_t = jnp.transpose(gt_boxes, (0, 2, 1))
    nt = H // HT
    out = pl.pallas_call(
        _loss_kernel,
        grid=(nt,),
        in_specs=[
            pl.BlockSpec((C, HT, B, W), lambda t: (0, t, 0, 0)),
            pl.BlockSpec((7, HT, B, W), lambda t: (0, t, 0, 0)),
            pl.BlockSpec((B, HT, 2, W), lambda t: (0, t, 0, 0)),
            pl.BlockSpec((B, 8, N), lambda t: (0, 0, 0)),
        ],
        out_specs=pl.BlockSpec((1, 1, 4), lambda t: (0, 0, 0)),
        out_shape=jax.ShapeDtypeStruct((1, 1, 4), jnp.float32),
        scratch_shapes=[
            pltpu.VMEM((B, 9, W, N), jnp.float32),
            pltpu.VMEM((1, 128), jnp.float32),
        ],
        compiler_params=pltpu.CompilerParams(
            dimension_semantics=("arbitrary",),
        ),
    )(cls_t, reg_t, dir_t, gt_t)
    return out.reshape(4)


# speedup vs baseline: 1.2447x; 1.2447x over previous
import jax
import jax.numpy as jnp
from jax.experimental import pallas as pl
from jax.experimental.pallas import tpu as pltpu

X_MIN, X_MAX = 0.0, 200.0
Y_MIN, Y_MAX = -50.0, 50.0
S = 0.4
INV_S = 2.5
ALPHA = 0.25
W_CLS, W_REG, W_DIR = 1.0, 2.0, 0.2
CW = (1.0, 5.0, 5.0)
HT = 25
H_FULL = 250
NP = 128


def _ls_pair(x):
    sp = jnp.log1p(jnp.exp(-jnp.abs(x)))
    ls_pos = jnp.where(x >= 0, -sp, x - sp)
    return ls_pos, ls_pos - x


def _pad_boxes(a, fill):
    return jnp.concatenate([a, jnp.full_like(a, fill)], axis=1)


def _slab_copies(cls_hbm, reg_hbm, dir_hbm, cbuf, rbuf, dbuf, sem, step, slot):
    rb = step * HT
    cps = []
    for c in range(3):
        for b in range(8):
            cps.append(pltpu.make_async_copy(
                cls_hbm.at[c, pl.ds(rb, HT), b, :], cbuf.at[slot, c, b],
                sem.at[slot]))
    for i in range(7):
        for b in range(8):
            cps.append(pltpu.make_async_copy(
                reg_hbm.at[i, pl.ds(rb, HT), b, :], rbuf.at[slot, i, b],
                sem.at[slot]))
    for k in range(2):
        for b in range(8):
            cps.append(pltpu.make_async_copy(
                dir_hbm.at[b, pl.ds(rb, HT), k, :], dbuf.at[slot, k, b],
                sem.at[slot]))
    return cps


def _loss_kernel(cls_hbm, reg_hbm, dir_hbm, gt_ref, out_ref,
                 cbuf, rbuf, dbuf, cmask_ref, acc_ref, sem):
    f32 = jnp.float32
    B = gt_ref.shape[0]
    W = cls_hbm.shape[3]
    t = pl.program_id(0)
    nt = pl.num_programs(0)
    slot = jax.lax.rem(t, 2)

    @pl.when(t == 0)
    def _():
        for cp in _slab_copies(cls_hbm, reg_hbm, dir_hbm,
                               cbuf, rbuf, dbuf, sem, 0, 0):
            cp.start()

    @pl.when(t + 1 < nt)
    def _():
        for cp in _slab_copies(cls_hbm, reg_hbm, dir_hbm,
                               cbuf, rbuf, dbuf, sem, t + 1, 1 - slot):
            cp.start()

    dn_rows = (((1,), (1,)), ((), ()))
    dn_gram = (((0,), (0,)), ((), ()))

    def dot_nt(a, b):
        return jax.lax.dot_general(a, b, dn_rows, preferred_element_type=f32)

    def gram(a, b):
        return jax.lax.dot_general(a, b, dn_gram, preferred_element_type=f32)

    gt = gt_ref[...]
    x, y = gt[:, 0, :], gt[:, 1, :]
    rot = gt[:, 6, :]
    gxf = jnp.floor((x - X_MIN) * INV_S)
    gyf = jnp.floor((y - Y_MIN) * INV_S)
    valid = ((x >= X_MIN) & (x < X_MAX) & (y >= Y_MIN) & (y < Y_MAX)
             & (gxf >= 0) & (gxf < W) & (gyf >= 0) & (gyf < H_FULL))
    gx = _pad_boxes(gxf.astype(jnp.int32), -7)
    gy = _pad_boxes(gyf.astype(jnp.int32), -7)
    cid = _pad_boxes(gt[:, 7, :].astype(jnp.int32), -1)
    validp = _pad_boxes(jnp.where(valid, 1, 0), 0) == 1
    dbin0 = _pad_boxes(jnp.where(jnp.cos(rot) >= 0.0, 1, 0), 0) == 1

    @pl.when(t == 0)
    def _():
        z3, l3, w3, h3 = gt[:, 2, :], gt[:, 3, :], gt[:, 4, :], gt[:, 5, :]
        rv_all = tuple(_pad_boxes(r, 0.0) for r in (
            (x - (X_MIN + (gxf + 0.5) * S)) * INV_S,
            (y - (Y_MIN + (gyf + 0.5) * S)) * INV_S, z3,
            jnp.log(jnp.maximum(l3, 1e-3)),
            jnp.log(jnp.maximum(w3, 1e-3)),
            jnp.log(jnp.maximum(h3, 1e-3)), jnp.sin(rot)))
        col_io = jax.lax.broadcasted_iota(jnp.int32, (W, NP), 0)
        row_io = jax.lax.broadcasted_iota(jnp.int32, (H_FULL, NP), 0)
        later = jax.lax.broadcasted_iota(jnp.int32, (NP, NP), 0) > \
            jax.lax.broadcasted_iota(jnp.int32, (NP, NP), 1)
        for b in range(B):
            gxb, gyb, vb = gx[b:b + 1, :], gy[b:b + 1, :], validp[b:b + 1, :]
            cols_eq = col_io == gxb
            cols_hit = jnp.where(cols_eq & vb, 1.0, 0.0).astype(f32)
            cmask_ref[b, 0] = cols_hit
            cmask_ref[b, 1] = jnp.where((jnp.abs(col_io - gxb) <= 1) & vb,
                                        1.0, 0.0).astype(f32)
            rows_oh = jnp.where((row_io == gyb) & vb, 1.0, 0.0).astype(f32)
            same = (gram(rows_oh, rows_oh) > 0.0) & \
                (gram(cols_hit, cols_hit) > 0.0)
            killed = jnp.any(same & later, axis=0, keepdims=True)
            islast = vb & jnp.logical_not(killed)
            keep = cols_eq & islast
            for i in range(7):
                cmask_ref[b, 2 + i] = jnp.where(keep, rv_all[i][b:b + 1, :],
                                                0.0).astype(f32)
        acc_ref[...] = jnp.zeros_like(acc_ref)

    for cp in _slab_copies(cls_hbm, reg_hbm, dir_hbm,
                           cbuf, rbuf, dbuf, sem, t, slot):
        cp.wait()

    row_io_t = jax.lax.broadcasted_iota(jnp.int32, (HT, NP), 0)
    rbase = t * HT

    cls_acc = jnp.zeros((), f32)
    vm_cnt = jnp.zeros((), f32)
    reg_num = jnp.zeros((), f32)
    pos_cnt = jnp.zeros((), f32)
    dir_acc = jnp.zeros((), f32)

    for b in range(B):
        gy_rel = gy[b:b + 1, :] - rbase
        rows_eq = row_io_t == gy_rel
        rows_near = jnp.abs(row_io_t - gy_rel) <= 1
        cid_b = cid[b:b + 1, :]
        cols_hit = cmask_ref[b, 0]

        poscnt = None
        for c in range(3):
            mc = cid_b == c
            hitcnt = dot_nt(jnp.where(rows_eq & mc, 1.0, 0.0).astype(f32),
                            cols_hit)
            nearcnt = dot_nt(jnp.where(rows_near & mc, 1.0, 0.0).astype(f32),
                             cmask_ref[b, 1])
            hit = hitcnt > 0.0
            vm = hit | (nearcnt == 0.0)
            xl = cbuf[slot, c, b]
            ls_pos, ls_neg = _ls_pair(xl)
            ls_a = jnp.where(hit, ls_pos, ls_neg)
            ls_b = jnp.where(hit, ls_neg, ls_pos)
            fac = jnp.where(hit, ALPHA * CW[c], 1.0 - ALPHA)
            term = fac * jnp.exp(3.0 * ls_b) * ls_a
            cls_acc = cls_acc + jnp.sum(jnp.where(vm, term, 0.0))
            vm_cnt = vm_cnt + jnp.sum(jnp.where(vm, 1.0, 0.0))
            poscnt = hitcnt if poscnt is None else poscnt + hitcnt

        pos = poscnt > 0.0
        pos_cnt = pos_cnt + jnp.sum(jnp.where(pos, 1.0, 0.0))

        rows_oh = jnp.where(rows_eq, 1.0, 0.0).astype(f32)
        for i in range(7):
            reg_t = dot_nt(rows_oh, cmask_ref[b, 2 + i])
            d = rbuf[slot, i, b] - reg_t
            ad = jnp.abs(d)
            sl1 = jnp.where(ad < 1.0, 0.5 * d * d, ad - 0.5)
            reg_num = reg_num + jnp.sum(jnp.where(pos, sl1, 0.0))

        for k in range(2):
            mk = dbin0[b:b + 1, :] if k == 0 else \
                jnp.logical_not(dbin0[b:b + 1, :])
            tk = dot_nt(jnp.where(rows_eq & mk & validp[b:b + 1, :],
                                  1.0, 0.0).astype(f32), cols_hit) > 0.0
            ls_pos, ls_neg = _ls_pair(dbuf[slot, k, b])
            dsel = jnp.where(tk, ls_pos, ls_neg)
            dir_acc = dir_acc + jnp.sum(jnp.where(pos, dsel, 0.0))

    lane = jax.lax.broadcasted_iota(jnp.int32, (1, 128), 1)
    vec = (jnp.where(lane == 0, cls_acc, 0.0)
           + jnp.where(lane == 1, vm_cnt, 0.0)
           + jnp.where(lane == 2, reg_num, 0.0)
           + jnp.where(lane == 3, pos_cnt, 0.0)
           + jnp.where(lane == 4, dir_acc, 0.0)).astype(f32)
    acc_ref[...] = acc_ref[...] + vec

    @pl.when(t == nt - 1)
    def _():
        s = acc_ref[...]
        cls_loss = -s[0, 0] / (s[0, 1] + 1e-6)
        rcnt = s[0, 3] * 7.0
        reg_loss = jnp.where(rcnt > 0, s[0, 2] / jnp.maximum(rcnt, 1.0), 0.0)
        dcnt = s[0, 3] * 2.0
        dir_loss = jnp.where(dcnt > 0, -s[0, 4] / jnp.maximum(dcnt, 1.0), 0.0)
        total = W_CLS * cls_loss + W_REG * reg_loss + W_DIR * dir_loss
        l4 = jax.lax.broadcasted_iota(jnp.int32, (1, 1, 4), 2)
        out_ref[...] = jnp.where(
            l4 == 0, total,
            jnp.where(l4 == 1, cls_loss,
                      jnp.where(l4 == 2, reg_loss, dir_loss))).astype(f32)


def kernel(cls_pred, reg_pred, dir_pred, gt_boxes, batch_size):
    B, C, H, W = cls_pred.shape
    N = gt_boxes.shape[1]
    cls_t = jnp.transpose(cls_pred, (1, 2, 0, 3))
    reg_t = jnp.transpose(reg_pred, (1, 2, 0, 3))
    dir_t = jnp.transpose(dir_pred, (0, 2, 1, 3))
    gt_t = jnp.transpose(gt_boxes, (0, 2, 1))
    nt = H // HT
    out = pl.pallas_call(
        _loss_kernel,
        grid=(nt,),
        in_specs=[
            pl.BlockSpec(memory_space=pl.ANY),
            pl.BlockSpec(memory_space=pl.ANY),
            pl.BlockSpec(memory_space=pl.ANY),
            pl.BlockSpec((B, 8, N), lambda t: (0, 0, 0)),
        ],
        out_specs=pl.BlockSpec((1, 1, 4), lambda t: (0, 0, 0)),
        out_shape=jax.ShapeDtypeStruct((1, 1, 4), jnp.float32),
        scratch_shapes=[
            pltpu.VMEM((2, C, B, HT, W), jnp.float32),
            pltpu.VMEM((2, 7, B, HT, W), jnp.float32),
            pltpu.VMEM((2, 2, B, HT, W), jnp.float32),
            pltpu.VMEM((B, 9, W, NP), jnp.float32),
            pltpu.VMEM((1, 128), jnp.float32),
            pltpu.SemaphoreType.DMA((2,)),
        ],
        compiler_params=pltpu.CompilerParams(
            dimension_semantics=("arbitrary",),
            vmem_limit_bytes=50 * 1024 * 1024,
        ),
    )(cls_t, reg_t, dir_t, gt_t)
    return out.reshape(4)


# speedup vs baseline: 1.8233x; 1.4648x over previous
import jax
import jax.numpy as jnp
from jax.experimental import pallas as pl
from jax.experimental.pallas import tpu as pltpu

X_MIN, X_MAX = 0.0, 200.0
Y_MIN, Y_MAX = -50.0, 50.0
S = 0.4
INV_S = 2.5
ALPHA = 0.25
W_CLS, W_REG, W_DIR = 1.0, 2.0, 0.2
CW = (1.0, 5.0, 5.0)
HT = 50
H_FULL = 250
NP = 128


def _ls_pair(x):
    sp = jnp.log1p(jnp.exp(-jnp.abs(x)))
    ls_pos = jnp.where(x >= 0, -sp, x - sp)
    return ls_pos, ls_pos - x


def _pad_boxes(a, fill):
    return jnp.concatenate([a, jnp.full_like(a, fill)], axis=1)


def _slab_copies(cls_hbm, reg_hbm, dir_hbm, cbuf, rbuf, dbuf, sem, step, slot):
    rb = step * HT
    cps = []
    for c in range(3):
        for b in range(8):
            cps.append(pltpu.make_async_copy(
                cls_hbm.at[c, pl.ds(rb, HT), b, :], cbuf.at[slot, c, b],
                sem.at[slot]))
    for i in range(7):
        for b in range(8):
            cps.append(pltpu.make_async_copy(
                reg_hbm.at[i, pl.ds(rb, HT), b, :], rbuf.at[slot, i, b],
                sem.at[slot]))
    for k in range(2):
        for b in range(8):
            cps.append(pltpu.make_async_copy(
                dir_hbm.at[b, pl.ds(rb, HT), k, :], dbuf.at[slot, k, b],
                sem.at[slot]))
    return cps


def _loss_kernel(cls_hbm, reg_hbm, dir_hbm, gt_ref, out_ref,
                 cbuf, rbuf, dbuf, cmask_ref, acc_ref, sem):
    f32 = jnp.float32
    B = gt_ref.shape[0]
    W = cls_hbm.shape[3]
    t = pl.program_id(0)
    nt = pl.num_programs(0)
    slot = jax.lax.rem(t, 2)

    @pl.when(t == 0)
    def _():
        for cp in _slab_copies(cls_hbm, reg_hbm, dir_hbm,
                               cbuf, rbuf, dbuf, sem, 0, 0):
            cp.start()

    @pl.when(t + 1 < nt)
    def _():
        for cp in _slab_copies(cls_hbm, reg_hbm, dir_hbm,
                               cbuf, rbuf, dbuf, sem, t + 1, 1 - slot):
            cp.start()

    dn_rows = (((1,), (1,)), ((), ()))
    dn_gram = (((0,), (0,)), ((), ()))

    def dot_nt(a, b):
        return jax.lax.dot_general(a, b, dn_rows, preferred_element_type=f32)

    def gram(a, b):
        return jax.lax.dot_general(a, b, dn_gram, preferred_element_type=f32)

    gt = gt_ref[...]
    x, y = gt[:, 0, :], gt[:, 1, :]
    rot = gt[:, 6, :]
    gxf = jnp.floor((x - X_MIN) * INV_S)
    gyf = jnp.floor((y - Y_MIN) * INV_S)
    valid = ((x >= X_MIN) & (x < X_MAX) & (y >= Y_MIN) & (y < Y_MAX)
             & (gxf >= 0) & (gxf < W) & (gyf >= 0) & (gyf < H_FULL))
    gx = _pad_boxes(gxf.astype(jnp.int32), -7)
    gy = _pad_boxes(gyf.astype(jnp.int32), -7)
    cid = _pad_boxes(gt[:, 7, :].astype(jnp.int32), -1)
    validp = _pad_boxes(jnp.where(valid, 1, 0), 0) == 1
    dbin0 = _pad_boxes(jnp.where(jnp.cos(rot) >= 0.0, 1, 0), 0) == 1

    @pl.when(t == 0)
    def _():
        z3, l3, w3, h3 = gt[:, 2, :], gt[:, 3, :], gt[:, 4, :], gt[:, 5, :]
        rv_all = tuple(_pad_boxes(r, 0.0) for r in (
            (x - (X_MIN + (gxf + 0.5) * S)) * INV_S,
            (y - (Y_MIN + (gyf + 0.5) * S)) * INV_S, z3,
            jnp.log(jnp.maximum(l3, 1e-3)),
            jnp.log(jnp.maximum(w3, 1e-3)),
            jnp.log(jnp.maximum(h3, 1e-3)), jnp.sin(rot)))
        col_io = jax.lax.broadcasted_iota(jnp.int32, (W, NP), 0)
        row_io = jax.lax.broadcasted_iota(jnp.int32, (H_FULL, NP), 0)
        later = jax.lax.broadcasted_iota(jnp.int32, (NP, NP), 0) > \
            jax.lax.broadcasted_iota(jnp.int32, (NP, NP), 1)
        for b in range(B):
            gxb, gyb, vb = gx[b:b + 1, :], gy[b:b + 1, :], validp[b:b + 1, :]
            cols_eq = col_io == gxb
            cols_hit = jnp.where(cols_eq & vb, 1.0, 0.0).astype(f32)
            cmask_ref[b, 0] = cols_hit
            cmask_ref[b, 1] = jnp.where((jnp.abs(col_io - gxb) <= 1) & vb,
                                        1.0, 0.0).astype(f32)
            rows_oh = jnp.where((row_io == gyb) & vb, 1.0, 0.0).astype(f32)
            same = (gram(rows_oh, rows_oh) > 0.0) & \
                (gram(cols_hit, cols_hit) > 0.0)
            killed = jnp.any(same & later, axis=0, keepdims=True)
            islast = vb & jnp.logical_not(killed)
            keep = cols_eq & islast
            for i in range(7):
                cmask_ref[b, 2 + i] = jnp.where(keep, rv_all[i][b:b + 1, :],
                                                0.0).astype(f32)
        acc_ref[...] = jnp.zeros_like(acc_ref)

    for cp in _slab_copies(cls_hbm, reg_hbm, dir_hbm,
                           cbuf, rbuf, dbuf, sem, t, slot):
        cp.wait()

    row_io_t = jax.lax.broadcasted_iota(jnp.int32, (HT, NP), 0)
    rbase = t * HT

    cls_acc = jnp.zeros((), f32)
    vm_cnt = jnp.zeros((), f32)
    reg_num = jnp.zeros((), f32)
    pos_cnt = jnp.zeros((), f32)
    dir_acc = jnp.zeros((), f32)

    for b in range(B):
        gy_rel = gy[b:b + 1, :] - rbase
        rows_eq = row_io_t == gy_rel
        rows_near = jnp.abs(row_io_t - gy_rel) <= 1
        cid_b = cid[b:b + 1, :]
        cols_hit = cmask_ref[b, 0]

        poscnt = None
        for c in range(3):
            mc = cid_b == c
            hitcnt = dot_nt(jnp.where(rows_eq & mc, 1.0, 0.0).astype(f32),
                            cols_hit)
            nearcnt = dot_nt(jnp.where(rows_near & mc, 1.0, 0.0).astype(f32),
                             cmask_ref[b, 1])
            hit = hitcnt > 0.0
            vm = hit | (nearcnt == 0.0)
            xl = cbuf[slot, c, b]
            ls_pos, ls_neg = _ls_pair(xl)
            ls_a = jnp.where(hit, ls_pos, ls_neg)
            ls_b = jnp.where(hit, ls_neg, ls_pos)
            fac = jnp.where(hit, ALPHA * CW[c], 1.0 - ALPHA)
            term = fac * jnp.exp(3.0 * ls_b) * ls_a
            cls_acc = cls_acc + jnp.sum(jnp.where(vm, term, 0.0))
            vm_cnt = vm_cnt + jnp.sum(jnp.where(vm, 1.0, 0.0))
            poscnt = hitcnt if poscnt is None else poscnt + hitcnt

        pos = poscnt > 0.0
        pos_cnt = pos_cnt + jnp.sum(jnp.where(pos, 1.0, 0.0))

        rows_oh = jnp.where(rows_eq, 1.0, 0.0).astype(f32)
        for i in range(7):
            reg_t = dot_nt(rows_oh, cmask_ref[b, 2 + i])
            d = rbuf[slot, i, b] - reg_t
            ad = jnp.abs(d)
            sl1 = jnp.where(ad < 1.0, 0.5 * d * d, ad - 0.5)
            reg_num = reg_num + jnp.sum(jnp.where(pos, sl1, 0.0))

        for k in range(2):
            mk = dbin0[b:b + 1, :] if k == 0 else \
                jnp.logical_not(dbin0[b:b + 1, :])
            tk = dot_nt(jnp.where(rows_eq & mk & validp[b:b + 1, :],
                                  1.0, 0.0).astype(f32), cols_hit) > 0.0
            ls_pos, ls_neg = _ls_pair(dbuf[slot, k, b])
            dsel = jnp.where(tk, ls_pos, ls_neg)
            dir_acc = dir_acc + jnp.sum(jnp.where(pos, dsel, 0.0))

    lane = jax.lax.broadcasted_iota(jnp.int32, (1, 128), 1)
    vec = (jnp.where(lane == 0, cls_acc, 0.0)
           + jnp.where(lane == 1, vm_cnt, 0.0)
           + jnp.where(lane == 2, reg_num, 0.0)
           + jnp.where(lane == 3, pos_cnt, 0.0)
           + jnp.where(lane == 4, dir_acc, 0.0)).astype(f32)
    acc_ref[...] = acc_ref[...] + vec

    @pl.when(t == nt - 1)
    def _():
        s = acc_ref[...]
        cls_loss = -s[0, 0] / (s[0, 1] + 1e-6)
        rcnt = s[0, 3] * 7.0
        reg_loss = jnp.where(rcnt > 0, s[0, 2] / jnp.maximum(rcnt, 1.0), 0.0)
        dcnt = s[0, 3] * 2.0
        dir_loss = jnp.where(dcnt > 0, -s[0, 4] / jnp.maximum(dcnt, 1.0), 0.0)
        total = W_CLS * cls_loss + W_REG * reg_loss + W_DIR * dir_loss
        l4 = jax.lax.broadcasted_iota(jnp.int32, (1, 1, 4), 2)
        out_ref[...] = jnp.where(
            l4 == 0, total,
            jnp.where(l4 == 1, cls_loss,
                      jnp.where(l4 == 2, reg_loss, dir_loss))).astype(f32)


def kernel(cls_pred, reg_pred, dir_pred, gt_boxes, batch_size):
    B, C, H, W = cls_pred.shape
    N = gt_boxes.shape[1]
    cls_t = jnp.transpose(cls_pred, (1, 2, 0, 3))
    reg_t = jnp.transpose(reg_pred, (1, 2, 0, 3))
    dir_t = jnp.transpose(dir_pred, (0, 2, 1, 3))
    gt_t = jnp.transpose(gt_boxes, (0, 2, 1))
    nt = H // HT
    out = pl.pallas_call(
        _loss_kernel,
        grid=(nt,),
        in_specs=[
            pl.BlockSpec(memory_space=pl.ANY),
            pl.BlockSpec(memory_space=pl.ANY),
            pl.BlockSpec(memory_space=pl.ANY),
            pl.BlockSpec((B, 8, N), lambda t: (0, 0, 0)),
        ],
        out_specs=pl.BlockSpec((1, 1, 4), lambda t: (0, 0, 0)),
        out_shape=jax.ShapeDtypeStruct((1, 1, 4), jnp.float32),
        scratch_shapes=[
            pltpu.VMEM((2, C, B, HT, W), jnp.float32),
            pltpu.VMEM((2, 7, B, HT, W), jnp.float32),
            pltpu.VMEM((2, 2, B, HT, W), jnp.float32),
            pltpu.VMEM((B, 9, W, NP), jnp.float32),
            pltpu.VMEM((1, 128), jnp.float32),
            pltpu.SemaphoreType.DMA((2,)),
        ],
        compiler_params=pltpu.CompilerParams(
            dimension_semantics=("arbitrary",),
            vmem_limit_bytes=56 * 1024 * 1024,
        ),
    )(cls_t, reg_t, dir_t, gt_t)
    return out.reshape(4)


# speedup vs baseline: 2.7072x; 1.4848x over previous
import jax
import jax.numpy as jnp
from jax.experimental import pallas as pl
from jax.experimental.pallas import tpu as pltpu

X_MIN, X_MAX = 0.0, 200.0
Y_MIN, Y_MAX = -50.0, 50.0
S = 0.4
INV_S = 2.5
ALPHA = 0.25
W_CLS, W_REG, W_DIR = 1.0, 2.0, 0.2
CW = (1.0, 5.0, 5.0)
HT = 50
H_FULL = 250
NP = 128


def _ls_pair(x):
    sp = jnp.log1p(jnp.exp(-jnp.abs(x)))
    ls_pos = jnp.where(x >= 0, -sp, x - sp)
    return ls_pos, ls_pos - x


def _pad_boxes(a, fill):
    return jnp.concatenate([a, jnp.full_like(a, fill)], axis=1)


def _slab_copies(cls_hbm, reg_hbm, dir_hbm, cbuf, rbuf, dbuf, sem, step, slot):
    rb = step * HT
    cps = []
    for c in range(3):
        for b in range(8):
            cps.append(pltpu.make_async_copy(
                cls_hbm.at[c, pl.ds(rb, HT), b, :], cbuf.at[slot, c, b],
                sem.at[slot]))
    for i in range(7):
        for b in range(8):
            cps.append(pltpu.make_async_copy(
                reg_hbm.at[i, pl.ds(rb, HT), b, :], rbuf.at[slot, i, b],
                sem.at[slot]))
    for k in range(2):
        for b in range(8):
            cps.append(pltpu.make_async_copy(
                dir_hbm.at[b, pl.ds(rb, HT), k, :], dbuf.at[slot, k, b],
                sem.at[slot]))
    return cps


def _loss_kernel(cls_hbm, reg_hbm, dir_hbm, gt_ref, out_ref,
                 cbuf, rbuf, dbuf, cmask_ref, boxv_ref, acc_ref, sem):
    f32 = jnp.float32
    B = gt_ref.shape[0]
    W = cls_hbm.shape[3]
    t = pl.program_id(0)
    nt = pl.num_programs(0)
    slot = jax.lax.rem(t, 2)

    @pl.when(t == 0)
    def _():
        for cp in _slab_copies(cls_hbm, reg_hbm, dir_hbm,
                               cbuf, rbuf, dbuf, sem, 0, 0):
            cp.start()

    @pl.when(t + 1 < nt)
    def _():
        for cp in _slab_copies(cls_hbm, reg_hbm, dir_hbm,
                               cbuf, rbuf, dbuf, sem, t + 1, 1 - slot):
            cp.start()

    dn_rows = (((1,), (1,)), ((), ()))
    dn_gram = (((0,), (0,)), ((), ()))

    def dot_nt(a, b):
        return jax.lax.dot_general(a, b, dn_rows, preferred_element_type=f32)

    def gram(a, b):
        return jax.lax.dot_general(a, b, dn_gram, preferred_element_type=f32)

    gt = gt_ref[...]
    x, y = gt[:, 0, :], gt[:, 1, :]
    rot = gt[:, 6, :]
    gxf = jnp.floor((x - X_MIN) * INV_S)
    gyf = jnp.floor((y - Y_MIN) * INV_S)
    valid = ((x >= X_MIN) & (x < X_MAX) & (y >= Y_MIN) & (y < Y_MAX)
             & (gxf >= 0) & (gxf < W) & (gyf >= 0) & (gyf < H_FULL))
    gx = _pad_boxes(gxf.astype(jnp.int32), -7)
    gy = _pad_boxes(gyf.astype(jnp.int32), -7)
    cid = _pad_boxes(gt[:, 7, :].astype(jnp.int32), -1)
    validp = _pad_boxes(jnp.where(valid, 1, 0), 0) == 1
    dbin0 = _pad_boxes(jnp.where(jnp.cos(rot) >= 0.0, 1, 0), 0) == 1

    @pl.when(t == 0)
    def _():
        col_io = jax.lax.broadcasted_iota(jnp.int32, (W, NP), 0)
        row_io = jax.lax.broadcasted_iota(jnp.int32, (H_FULL, NP), 0)
        later = jax.lax.broadcasted_iota(jnp.int32, (NP, NP), 0) > \
            jax.lax.broadcasted_iota(jnp.int32, (NP, NP), 1)
        pos_total = jnp.zeros((), f32)
        for b in range(B):
            gxb, gyb, vb = gx[b:b + 1, :], gy[b:b + 1, :], validp[b:b + 1, :]
            db = dbin0[b:b + 1, :]
            cols_eq = col_io == gxb
            cols_hit = jnp.where(cols_eq & vb, 1.0, 0.0).astype(f32)
            cmask_ref[b, 0] = cols_hit
            cmask_ref[b, 1] = jnp.where((jnp.abs(col_io - gxb) <= 1) & vb,
                                        1.0, 0.0).astype(f32)
            rows_oh = jnp.where((row_io == gyb) & vb, 1.0, 0.0).astype(f32)
            same = (gram(rows_oh, rows_oh) > 0.0) & \
                (gram(cols_hit, cols_hit) > 0.0)
            killed = jnp.any(same & later, axis=0, keepdims=True)
            islast = vb & jnp.logical_not(killed)
            boxv_ref[b, 0] = jnp.where(islast, 1.0, 0.0).astype(f32)
            pos_total = pos_total + jnp.sum(jnp.where(islast, 1.0, 0.0))
            rows_d0 = jnp.where((row_io == gyb) & vb & db, 1.0,
                                0.0).astype(f32)
            cols_d0 = jnp.where(cols_eq & vb & db, 1.0, 0.0).astype(f32)
            t0 = (gram(rows_d0, rows_oh) > 0.0) & \
                (gram(cols_d0, cols_hit) > 0.0)
            rows_d1 = jnp.where((row_io == gyb) & vb & ~db, 1.0,
                                0.0).astype(f32)
            cols_d1 = jnp.where(cols_eq & vb & ~db, 1.0, 0.0).astype(f32)
            t1 = (gram(rows_d1, rows_oh) > 0.0) & \
                (gram(cols_d1, cols_hit) > 0.0)
            boxv_ref[b, 1] = jnp.where(jnp.any(t0, axis=0, keepdims=True),
                                       1.0, 0.0).astype(f32)
            boxv_ref[b, 2] = jnp.where(jnp.any(t1, axis=0, keepdims=True),
                                       1.0, 0.0).astype(f32)
        acc_ref[...] = jnp.zeros_like(acc_ref)
        lane0 = jax.lax.broadcasted_iota(jnp.int32, (1, 128), 1)
        acc_ref[...] = acc_ref[...] + jnp.where(lane0 == 3, pos_total, 0.0)

    for cp in _slab_copies(cls_hbm, reg_hbm, dir_hbm,
                           cbuf, rbuf, dbuf, sem, t, slot):
        cp.wait()

    row_io_t = jax.lax.broadcasted_iota(jnp.int32, (HT, NP), 0)
    rbase = t * HT

    z3, l3, w3, h3 = gt[:, 2, :], gt[:, 3, :], gt[:, 4, :], gt[:, 5, :]
    rv_all = tuple(_pad_boxes(r, 0.0) for r in (
        (x - (X_MIN + (gxf + 0.5) * S)) * INV_S,
        (y - (Y_MIN + (gyf + 0.5) * S)) * INV_S, z3,
        jnp.log(jnp.maximum(l3, 1e-3)),
        jnp.log(jnp.maximum(w3, 1e-3)),
        jnp.log(jnp.maximum(h3, 1e-3)), jnp.sin(rot)))

    dn_gather = (((1,), (0,)), ((), ()))

    def gather_mm(slab, cols):
        return jax.lax.dot_general(slab, cols, dn_gather,
                                   preferred_element_type=f32)

    cls_acc = jnp.zeros((), f32)
    vm_cnt = jnp.zeros((), f32)
    regvec = jnp.zeros((1, NP), f32)
    dirvec = jnp.zeros((1, NP), f32)

    for b in range(B):
        gy_rel = gy[b:b + 1, :] - rbase
        rows_eq = row_io_t == gy_rel
        rows_near = jnp.abs(row_io_t - gy_rel) <= 1
        cid_b = cid[b:b + 1, :]
        cols_hit = cmask_ref[b, 0]
        rows_oh = jnp.where(rows_eq, 1.0, 0.0).astype(f32)
        intile = (boxv_ref[b, 0] > 0.0) & (gy_rel >= 0) & (gy_rel < HT)

        for c in range(3):
            mc = cid_b == c
            hitcnt = dot_nt(jnp.where(rows_eq & mc, 1.0, 0.0).astype(f32),
                            cols_hit)
            nearcnt = dot_nt(jnp.where(rows_near & mc, 1.0, 0.0).astype(f32),
                             cmask_ref[b, 1])
            hit = hitcnt > 0.0
            vm = hit | (nearcnt == 0.0)
            xl = cbuf[slot, c, b]
            ls_pos, ls_neg = _ls_pair(xl)
            ls_a = jnp.where(hit, ls_pos, ls_neg)
            ls_b = jnp.where(hit, ls_neg, ls_pos)
            fac = jnp.where(hit, ALPHA * CW[c], 1.0 - ALPHA)
            term = fac * jnp.exp(3.0 * ls_b) * ls_a
            cls_acc = cls_acc + jnp.sum(jnp.where(vm, term, 0.0))
            vm_cnt = vm_cnt + jnp.sum(jnp.where(vm, 1.0, 0.0))

        for i in range(7):
            g = jnp.sum(gather_mm(rbuf[slot, i, b], cols_hit) * rows_oh,
                        axis=0, keepdims=True)
            d = g - rv_all[i][b:b + 1, :]
            ad = jnp.abs(d)
            sl1 = jnp.where(ad < 1.0, 0.5 * d * d, ad - 0.5)
            regvec = regvec + jnp.where(intile, sl1, 0.0)

        for k in range(2):
            g = jnp.sum(gather_mm(dbuf[slot, k, b], cols_hit) * rows_oh,
                        axis=0, keepdims=True)
            ls_pos, ls_neg = _ls_pair(g)
            tk = boxv_ref[b, 1 + k] > 0.0
            dsel = jnp.where(tk, ls_pos, ls_neg)
            dirvec = dirvec + jnp.where(intile, dsel, 0.0)

    lane = jax.lax.broadcasted_iota(jnp.int32, (1, 128), 1)
    vec = (jnp.where(lane == 0, cls_acc, 0.0)
           + jnp.where(lane == 1, vm_cnt, 0.0)
           + jnp.where(lane == 2, jnp.sum(regvec), 0.0)
           + jnp.where(lane == 4, jnp.sum(dirvec), 0.0)).astype(f32)
    acc_ref[...] = acc_ref[...] + vec

    @pl.when(t == nt - 1)
    def _():
        s = acc_ref[...]
        cls_loss = -s[0, 0] / (s[0, 1] + 1e-6)
        rcnt = s[0, 3] * 7.0
        reg_loss = jnp.where(rcnt > 0, s[0, 2] / jnp.maximum(rcnt, 1.0), 0.0)
        dcnt = s[0, 3] * 2.0
        dir_loss = jnp.where(dcnt > 0, -s[0, 4] / jnp.maximum(dcnt, 1.0), 0.0)
        total = W_CLS * cls_loss + W_REG * reg_loss + W_DIR * dir_loss
        l4 = jax.lax.broadcasted_iota(jnp.int32, (1, 1, 4), 2)
        out_ref[...] = jnp.where(
            l4 == 0, total,
            jnp.where(l4 == 1, cls_loss,
                      jnp.where(l4 == 2, reg_loss, dir_loss))).astype(f32)


def kernel(cls_pred, reg_pred, dir_pred, gt_boxes, batch_size):
    B, C, H, W = cls_pred.shape
    N = gt_boxes.shape[1]
    cls_t = jnp.transpose(cls_pred, (1, 2, 0, 3))
    reg_t = jnp.transpose(reg_pred, (1, 2, 0, 3))
    dir_t = jnp.transpose(dir_pred, (0, 2, 1, 3))
    gt_t = jnp.transpose(gt_boxes, (0, 2, 1))
    nt = H // HT
    out = pl.pallas_call(
        _loss_kernel,
        grid=(nt,),
        in_specs=[
            pl.BlockSpec(memory_space=pl.ANY),
            pl.BlockSpec(memory_space=pl.ANY),
            pl.BlockSpec(memory_space=pl.ANY),
            pl.BlockSpec((B, 8, N), lambda t: (0, 0, 0)),
        ],
        out_specs=pl.BlockSpec((1, 1, 4), lambda t: (0, 0, 0)),
        out_shape=jax.ShapeDtypeStruct((1, 1, 4), jnp.float32),
        scratch_shapes=[
            pltpu.VMEM((2, C, B, HT, W), jnp.float32),
            pltpu.VMEM((2, 7, B, HT, W), jnp.float32),
            pltpu.VMEM((2, 2, B, HT, W), jnp.float32),
            pltpu.VMEM((B, 2, W, NP), jnp.float32),
            pltpu.VMEM((B, 3, 1, NP), jnp.float32),
            pltpu.VMEM((1, 128), jnp.float32),
            pltpu.SemaphoreType.DMA((2,)),
        ],
        compiler_params=pltpu.CompilerParams(
            dimension_semantics=("arbitrary",),
            vmem_limit_bytes=48 * 1024 * 1024,
        ),
    )(cls_t, reg_t, dir_t, gt_t)
    return out.reshape(4)
